# baseline (device time: 47291 ns/iter reference)
import jax
import jax.numpy as jnp
from jax import lax
from jax.experimental import pallas as pl
from jax.experimental.pallas import tpu as pltpu

N_Z = 4


def kernel(partial, resid, gamma):
    m, d = resid.shape
    p2 = partial.reshape(m, d)
    g2 = gamma.reshape(1, d)

    def body(p_ref, r_ref, g_ref, out_ref, comm_ref, send_sems, recv_sems):
        my_x = lax.axis_index("x")
        my_y = lax.axis_index("y")
        my_z = lax.axis_index("z")
        right = (my_z + 1) % N_Z
        left = (my_z + N_Z - 1) % N_Z

        barrier_sem = pltpu.get_barrier_semaphore()
        for nbr in (left, right):
            pl.semaphore_signal(
                barrier_sem,
                inc=1,
                device_id=(my_x, my_y, nbr),
                device_id_type=pl.DeviceIdType.MESH,
            )
        pl.semaphore_wait(barrier_sem, 2)

        comm_ref[0] = p_ref[...]
        for h in range(N_Z - 1):
            rdma = pltpu.make_async_remote_copy(
                src_ref=comm_ref.at[h],
                dst_ref=comm_ref.at[h + 1],
                send_sem=send_sems.at[h],
                recv_sem=recv_sems.at[h],
                device_id=(my_x, my_y, right),
                device_id_type=pl.DeviceIdType.MESH,
            )
            rdma.start()
            rdma.wait()

        y = (comm_ref[0] + comm_ref[1] + comm_ref[2] + comm_ref[3]) + r_ref[...]
        rms = jnp.sqrt(jnp.mean(y * y, axis=-1, keepdims=True) + 1e-6)
        out_ref[...] = y / rms * g_ref[...]

    return pl.pallas_call(
        body,
        out_shape=jax.ShapeDtypeStruct((m, d), jnp.float32),
        in_specs=[
            pl.BlockSpec(memory_space=pltpu.VMEM),
            pl.BlockSpec(memory_space=pltpu.VMEM),
            pl.BlockSpec(memory_space=pltpu.VMEM),
        ],
        out_specs=pl.BlockSpec(memory_space=pltpu.VMEM),
        scratch_shapes=[
            pltpu.VMEM((N_Z, m, d), jnp.float32),
            pltpu.SemaphoreType.DMA((N_Z - 1,)),
            pltpu.SemaphoreType.DMA((N_Z - 1,)),
        ],
        compiler_params=pltpu.CompilerParams(collective_id=0),
    )(p2, resid, g2)


# device time: 25701 ns/iter; 1.8400x vs baseline; 1.8400x over previous
import jax
import jax.numpy as jnp
from jax import lax
from jax.experimental import pallas as pl
from jax.experimental.pallas import tpu as pltpu

N_Z = 4
PIECE = 32
BLOCK = 128


def kernel(partial, resid, gamma):
    m, d = resid.shape
    p2 = partial.reshape(m, d)
    g2 = gamma.reshape(1, d)

    def body(
        p_ref, r_ref, g_ref, out_ref, rs_buf,
        p1_send, p1_recv, p2_send, p2_recv, p3_send, p3_recv,
    ):
        my_x = lax.axis_index("x")
        my_y = lax.axis_index("y")
        my_z = lax.axis_index("z")
        b = 2 * my_x + my_y
        block_start = BLOCK * b
        piece_start = block_start + PIECE * my_z

        z_peers = [(my_z + o) % N_Z for o in (1, 2, 3)]
        xy_peers = [(1 - my_x, my_y), (my_x, 1 - my_y), (1 - my_x, 1 - my_y)]

        barrier_sem = pltpu.get_barrier_semaphore()
        for t in z_peers:
            pl.semaphore_signal(
                barrier_sem, inc=1, device_id=(my_x, my_y, t),
                device_id_type=pl.DeviceIdType.MESH,
            )
        for tx, ty in xy_peers:
            pl.semaphore_signal(
                barrier_sem, inc=1, device_id=(tx, ty, my_z),
                device_id_type=pl.DeviceIdType.MESH,
            )
        pl.semaphore_wait(barrier_sem, 6)

        p1 = []
        for i, t in enumerate(z_peers):
            rdma = pltpu.make_async_remote_copy(
                src_ref=p_ref.at[pl.ds(block_start + PIECE * t, PIECE), :],
                dst_ref=rs_buf.at[i],
                send_sem=p1_send.at[i],
                recv_sem=p1_recv.at[i],
                device_id=(my_x, my_y, t),
                device_id_type=pl.DeviceIdType.MESH,
            )
            rdma.start()
            p1.append(rdma)
        for rdma in p1:
            rdma.wait()

        rows = pl.ds(piece_start, PIECE)
        y = p_ref[rows, :] + rs_buf[0] + rs_buf[1] + rs_buf[2] + r_ref[rows, :]
        rms = jnp.sqrt(jnp.mean(y * y, axis=-1, keepdims=True) + 1e-6)
        out_ref[rows, :] = y / rms * g_ref[...]

        p2_ops = []
        for i, t in enumerate(z_peers):
            rdma = pltpu.make_async_remote_copy(
                src_ref=out_ref.at[rows, :],
                dst_ref=out_ref.at[rows, :],
                send_sem=p2_send.at[i],
                recv_sem=p2_recv.at[i],
                device_id=(my_x, my_y, t),
                device_id_type=pl.DeviceIdType.MESH,
            )
            rdma.start()
            p2_ops.append(rdma)
        for rdma in p2_ops:
            rdma.wait()

        brows = pl.ds(block_start, BLOCK)
        p3_ops = []
        for i, (tx, ty) in enumerate(xy_peers):
            rdma = pltpu.make_async_remote_copy(
                src_ref=out_ref.at[brows, :],
                dst_ref=out_ref.at[brows, :],
                send_sem=p3_send.at[i],
                recv_sem=p3_recv.at[i],
                device_id=(tx, ty, my_z),
                device_id_type=pl.DeviceIdType.MESH,
            )
            rdma.start()
            p3_ops.append(rdma)
        for rdma in p3_ops:
            rdma.wait()

    return pl.pallas_call(
        body,
        out_shape=jax.ShapeDtypeStruct((m, d), jnp.float32),
        in_specs=[
            pl.BlockSpec(memory_space=pltpu.VMEM),
            pl.BlockSpec(memory_space=pltpu.VMEM),
            pl.BlockSpec(memory_space=pltpu.VMEM),
        ],
        out_specs=pl.BlockSpec(memory_space=pltpu.VMEM),
        scratch_shapes=[
            pltpu.VMEM((3, PIECE, d), jnp.float32),
            pltpu.SemaphoreType.DMA((3,)),
            pltpu.SemaphoreType.DMA((3,)),
            pltpu.SemaphoreType.DMA((3,)),
            pltpu.SemaphoreType.DMA((3,)),
            pltpu.SemaphoreType.DMA((3,)),
            pltpu.SemaphoreType.DMA((3,)),
        ],
        compiler_params=pltpu.CompilerParams(collective_id=0),
    )(p2, resid, g2)


# device time: 24385 ns/iter; 1.9393x vs baseline; 1.0540x over previous
import jax
import jax.numpy as jnp
from jax import lax
from jax.experimental import pallas as pl
from jax.experimental.pallas import tpu as pltpu

N_Z = 4
PIECE = 32
BLOCK = 128


def kernel(partial, resid, gamma):
    m, d = resid.shape
    p2 = partial.reshape(m, d)
    g2 = gamma.reshape(1, d)

    def body(
        p_ref, r_ref, g_ref, out_ref, rs_buf,
        p1_send, p1_recv, p2_send, p2_recv, p3_send, p3_recv,
    ):
        my_x = lax.axis_index("x")
        my_y = lax.axis_index("y")
        my_z = lax.axis_index("z")
        b = 2 * my_x + my_y
        block_start = BLOCK * b
        piece_start = block_start + PIECE * my_z

        z_peers = [(my_z + o) % N_Z for o in (1, 2, 3)]
        xy_peers = [(1 - my_x, my_y), (my_x, 1 - my_y), (1 - my_x, 1 - my_y)]

        barrier_sem = pltpu.get_barrier_semaphore()
        for t in z_peers:
            pl.semaphore_signal(
                barrier_sem, inc=1, device_id=(my_x, my_y, t),
                device_id_type=pl.DeviceIdType.MESH,
            )
        for tx, ty in xy_peers:
            pl.semaphore_signal(
                barrier_sem, inc=1, device_id=(tx, ty, my_z),
                device_id_type=pl.DeviceIdType.MESH,
            )
        pl.semaphore_wait(barrier_sem, 6)

        p1_ops = []
        for i, t in enumerate(z_peers):
            rdma = pltpu.make_async_remote_copy(
                src_ref=p_ref.at[pl.ds(block_start + PIECE * t, PIECE), :],
                dst_ref=rs_buf.at[i],
                send_sem=p1_send.at[i],
                recv_sem=p1_recv.at[i],
                device_id=(my_x, my_y, t),
                device_id_type=pl.DeviceIdType.MESH,
            )
            rdma.start()
            p1_ops.append(rdma)
        for rdma in p1_ops:
            rdma.wait_recv()

        rows = pl.ds(piece_start, PIECE)
        y = p_ref[rows, :] + rs_buf[0] + rs_buf[1] + rs_buf[2] + r_ref[rows, :]
        rms = jnp.sqrt(jnp.mean(y * y, axis=-1, keepdims=True) + 1e-6)
        out_ref[rows, :] = y / rms * g_ref[...]

        def plane_send(src_rows, k):
            ops = []
            for r, (tx, ty) in enumerate(xy_peers):
                rdma = pltpu.make_async_remote_copy(
                    src_ref=out_ref.at[src_rows, :],
                    dst_ref=out_ref.at[src_rows, :],
                    send_sem=p3_send.at[r, k],
                    recv_sem=p3_recv.at[r, k],
                    device_id=(tx, ty, my_z),
                    device_id_type=pl.DeviceIdType.MESH,
                )
                rdma.start()
                ops.append(rdma)
            return ops

        p2_ops = []
        for i, t in enumerate(z_peers):
            rdma = pltpu.make_async_remote_copy(
                src_ref=out_ref.at[rows, :],
                dst_ref=out_ref.at[rows, :],
                send_sem=p2_send.at[i],
                recv_sem=p2_recv.at[i],
                device_id=(my_x, my_y, t),
                device_id_type=pl.DeviceIdType.MESH,
            )
            rdma.start()
            p2_ops.append(rdma)

        p3_ops = plane_send(rows, 0)
        for j, rdma in enumerate(p2_ops):
            rdma.wait_recv()
            zz = (my_z - j - 1) % N_Z
            p3_ops += plane_send(pl.ds(block_start + PIECE * zz, PIECE), j + 1)

        for rdma in p3_ops:
            rdma.wait_recv()
        for rdma in p1_ops + p2_ops + p3_ops:
            rdma.wait_send()

    return pl.pallas_call(
        body,
        out_shape=jax.ShapeDtypeStruct((m, d), jnp.float32),
        in_specs=[
            pl.BlockSpec(memory_space=pltpu.VMEM),
            pl.BlockSpec(memory_space=pltpu.VMEM),
            pl.BlockSpec(memory_space=pltpu.VMEM),
        ],
        out_specs=pl.BlockSpec(memory_space=pltpu.VMEM),
        scratch_shapes=[
            pltpu.VMEM((3, PIECE, d), jnp.float32),
            pltpu.SemaphoreType.DMA((3,)),
            pltpu.SemaphoreType.DMA((3,)),
            pltpu.SemaphoreType.DMA((3,)),
            pltpu.SemaphoreType.DMA((3,)),
            pltpu.SemaphoreType.DMA((3, N_Z)),
            pltpu.SemaphoreType.DMA((3, N_Z)),
        ],
        compiler_params=pltpu.CompilerParams(collective_id=0),
    )(p2, resid, g2)


# device time: 22816 ns/iter; 2.0727x vs baseline; 1.0688x over previous
import jax
import jax.numpy as jnp
from jax import lax
from jax.experimental import pallas as pl
from jax.experimental.pallas import tpu as pltpu

N_Z = 4
PIECE = 32
BLOCK = 128


def kernel(partial, resid, gamma):
    m, d = resid.shape
    p2 = partial.reshape(m, d)
    g2 = gamma.reshape(1, d)

    mx = lax.axis_index("x")
    my = lax.axis_index("y")
    mz = lax.axis_index("z")
    blk0 = BLOCK * (2 * mx + my)
    p_blk = lax.dynamic_slice(p2, (blk0, 0), (BLOCK, d))
    r_pc = lax.dynamic_slice(resid, (blk0 + PIECE * mz, 0), (PIECE, d))

    def body(
        p_ref, r_ref, g_ref, out_ref, rs_buf,
        p1_send, p1_recv, p2_send, p2_recv, p3_send, p3_recv,
    ):
        my_x = lax.axis_index("x")
        my_y = lax.axis_index("y")
        my_z = lax.axis_index("z")
        b = 2 * my_x + my_y
        block_start = BLOCK * b
        piece_start = block_start + PIECE * my_z

        z_peers = [(my_z + o) % N_Z for o in (1, 2, 3)]
        xy_peers = [(1 - my_x, my_y), (my_x, 1 - my_y), (1 - my_x, 1 - my_y)]

        barrier_sem = pltpu.get_barrier_semaphore()
        for t in z_peers:
            pl.semaphore_signal(
                barrier_sem, inc=1, device_id=(my_x, my_y, t),
                device_id_type=pl.DeviceIdType.MESH,
            )
        for tx, ty in xy_peers:
            pl.semaphore_signal(
                barrier_sem, inc=1, device_id=(tx, ty, my_z),
                device_id_type=pl.DeviceIdType.MESH,
            )
        pl.semaphore_wait(barrier_sem, 6)

        p1_ops = []
        for i, t in enumerate(z_peers):
            rdma = pltpu.make_async_remote_copy(
                src_ref=p_ref.at[pl.ds(PIECE * t, PIECE), :],
                dst_ref=rs_buf.at[i],
                send_sem=p1_send.at[i],
                recv_sem=p1_recv.at[i],
                device_id=(my_x, my_y, t),
                device_id_type=pl.DeviceIdType.MESH,
            )
            rdma.start()
            p1_ops.append(rdma)
        for rdma in p1_ops:
            rdma.wait_recv()

        my_rows = pl.ds(PIECE * my_z, PIECE)
        rows = pl.ds(piece_start, PIECE)
        y = p_ref[my_rows, :] + rs_buf[0] + rs_buf[1] + rs_buf[2] + r_ref[...]
        rms = jnp.sqrt(jnp.mean(y * y, axis=-1, keepdims=True) + 1e-6)
        out_ref[rows, :] = y / rms * g_ref[...]

        def plane_send(src_rows, k):
            ops = []
            for r, (tx, ty) in enumerate(xy_peers):
                rdma = pltpu.make_async_remote_copy(
                    src_ref=out_ref.at[src_rows, :],
                    dst_ref=out_ref.at[src_rows, :],
                    send_sem=p3_send.at[r, k],
                    recv_sem=p3_recv.at[r, k],
                    device_id=(tx, ty, my_z),
                    device_id_type=pl.DeviceIdType.MESH,
                )
                rdma.start()
                ops.append(rdma)
            return ops

        p2_ops = []
        for i, t in enumerate(z_peers):
            rdma = pltpu.make_async_remote_copy(
                src_ref=out_ref.at[rows, :],
                dst_ref=out_ref.at[rows, :],
                send_sem=p2_send.at[i],
                recv_sem=p2_recv.at[i],
                device_id=(my_x, my_y, t),
                device_id_type=pl.DeviceIdType.MESH,
            )
            rdma.start()
            p2_ops.append(rdma)

        p3_ops = plane_send(rows, 0)
        for j, rdma in enumerate(p2_ops):
            rdma.wait_recv()
            zz = (my_z - j - 1) % N_Z
            p3_ops += plane_send(pl.ds(block_start + PIECE * zz, PIECE), j + 1)

        for rdma in p3_ops:
            rdma.wait_recv()
        for rdma in p1_ops + p2_ops:
            rdma.wait_send()
        for rdma in p3_ops:
            rdma.wait_send()

    return pl.pallas_call(
        body,
        out_shape=jax.ShapeDtypeStruct((m, d), jnp.float32),
        in_specs=[
            pl.BlockSpec(memory_space=pltpu.VMEM),
            pl.BlockSpec(memory_space=pltpu.VMEM),
            pl.BlockSpec(memory_space=pltpu.VMEM),
        ],
        out_specs=pl.BlockSpec(memory_space=pltpu.VMEM),
        scratch_shapes=[
            pltpu.VMEM((3, PIECE, d), jnp.float32),
            pltpu.SemaphoreType.DMA((3,)),
            pltpu.SemaphoreType.DMA((3,)),
            pltpu.SemaphoreType.DMA((3,)),
            pltpu.SemaphoreType.DMA((3,)),
            pltpu.SemaphoreType.DMA((3, N_Z)),
            pltpu.SemaphoreType.DMA((3, N_Z)),
        ],
        compiler_params=pltpu.CompilerParams(collective_id=0),
    )(p_blk, r_pc, g2)


# device time: 21417 ns/iter; 2.2081x vs baseline; 1.0653x over previous
import jax
import jax.numpy as jnp
from jax import lax
from jax.experimental import pallas as pl
from jax.experimental.pallas import tpu as pltpu

N_Z = 4
PIECE = 32
BLOCK = 128


def kernel(partial, resid, gamma):
    m, d = resid.shape
    p2 = partial.reshape(m, d)
    g2 = gamma.reshape(1, d)

    mx = lax.axis_index("x")
    my = lax.axis_index("y")
    mz = lax.axis_index("z")
    blk0 = BLOCK * (2 * mx + my)
    p_blk = lax.dynamic_slice(p2, (blk0, 0), (BLOCK, d))
    r_pc = lax.dynamic_slice(resid, (blk0 + PIECE * mz, 0), (PIECE, d))

    def body(
        p_ref, r_ref, g_ref, out_ref, rs_buf,
        p1_send, p1_recv, p2_send, p2_recv, p3_send, p3_recv,
    ):
        my_x = lax.axis_index("x")
        my_y = lax.axis_index("y")
        my_z = lax.axis_index("z")
        b = 2 * my_x + my_y
        block_start = BLOCK * b
        piece_start = block_start + PIECE * my_z

        z_peers = [(my_z + o) % N_Z for o in (1, 2, 3)]
        xy_peers = [(1 - my_x, my_y), (my_x, 1 - my_y), (1 - my_x, 1 - my_y)]

        barrier_sem = pltpu.get_barrier_semaphore()
        for t in z_peers:
            pl.semaphore_signal(
                barrier_sem, inc=1, device_id=(my_x, my_y, t),
                device_id_type=pl.DeviceIdType.MESH,
            )
        for tx, ty in xy_peers:
            pl.semaphore_signal(
                barrier_sem, inc=1, device_id=(tx, ty, my_z),
                device_id_type=pl.DeviceIdType.MESH,
            )
        pl.semaphore_wait(barrier_sem, 6)

        p1_ops = []
        for i, t in enumerate(z_peers):
            rdma = pltpu.make_async_remote_copy(
                src_ref=p_ref.at[pl.ds(PIECE * t, PIECE), :],
                dst_ref=rs_buf.at[i],
                send_sem=p1_send.at[i],
                recv_sem=p1_recv.at[i],
                device_id=(my_x, my_y, t),
                device_id_type=pl.DeviceIdType.MESH,
            )
            rdma.start()
            p1_ops.append(rdma)
        for rdma in p1_ops:
            rdma.wait_recv()

        my_rows = pl.ds(PIECE * my_z, PIECE)
        rows = pl.ds(piece_start, PIECE)
        y = p_ref[my_rows, :] + rs_buf[0] + rs_buf[1] + rs_buf[2] + r_ref[...]
        rms = jnp.sqrt(jnp.mean(y * y, axis=-1, keepdims=True) + 1e-6)
        out_ref[rows, :] = y / rms * g_ref[...]

        def plane_send(src_rows, k):
            ops = []
            for r, (tx, ty) in enumerate(xy_peers):
                rdma = pltpu.make_async_remote_copy(
                    src_ref=out_ref.at[src_rows, :],
                    dst_ref=out_ref.at[src_rows, :],
                    send_sem=p3_send.at[r, k],
                    recv_sem=p3_recv.at[r, k],
                    device_id=(tx, ty, my_z),
                    device_id_type=pl.DeviceIdType.MESH,
                )
                rdma.start()
                ops.append(rdma)
            return ops

        p2_ops = []
        for i, t in enumerate(z_peers):
            rdma = pltpu.make_async_remote_copy(
                src_ref=out_ref.at[rows, :],
                dst_ref=out_ref.at[rows, :],
                send_sem=p2_send.at[i],
                recv_sem=p2_recv.at[i],
                device_id=(my_x, my_y, t),
                device_id_type=pl.DeviceIdType.MESH,
            )
            rdma.start()
            p2_ops.append(rdma)

        p3_own = plane_send(rows, 0)

        ORDER = {0: (2, 1, 0), 1: (0, 2, 1), 2: (0, 2, 1), 3: (0, 1, 2)}
        for c in range(N_Z):
            @pl.when(my_z == c)
            def _(c=c):
                for k, j in enumerate(ORDER[c]):
                    p2_ops[j].wait_recv()
                    zz = (c - j - 1) % N_Z
                    plane_send(pl.ds(block_start + PIECE * zz, PIECE), k + 1)

        drains = []
        for k in range(1, N_Z):
            for r, (tx, ty) in enumerate(xy_peers):
                drains.append(pltpu.make_async_remote_copy(
                    src_ref=out_ref.at[rows, :],
                    dst_ref=out_ref.at[rows, :],
                    send_sem=p3_send.at[r, k],
                    recv_sem=p3_recv.at[r, k],
                    device_id=(tx, ty, my_z),
                    device_id_type=pl.DeviceIdType.MESH,
                ))
        for rdma in p3_own + drains:
            rdma.wait_recv()
        for rdma in p1_ops + p2_ops + p3_own + drains:
            rdma.wait_send()

    return pl.pallas_call(
        body,
        out_shape=jax.ShapeDtypeStruct((m, d), jnp.float32),
        in_specs=[
            pl.BlockSpec(memory_space=pltpu.VMEM),
            pl.BlockSpec(memory_space=pltpu.VMEM),
            pl.BlockSpec(memory_space=pltpu.VMEM),
        ],
        out_specs=pl.BlockSpec(memory_space=pltpu.VMEM),
        scratch_shapes=[
            pltpu.VMEM((3, PIECE, d), jnp.float32),
            pltpu.SemaphoreType.DMA((3,)),
            pltpu.SemaphoreType.DMA((3,)),
            pltpu.SemaphoreType.DMA((3,)),
            pltpu.SemaphoreType.DMA((3,)),
            pltpu.SemaphoreType.DMA((3, N_Z)),
            pltpu.SemaphoreType.DMA((3, N_Z)),
        ],
        compiler_params=pltpu.CompilerParams(collective_id=0),
    )(p_blk, r_pc, g2)
